# Initial kernel scaffold; baseline (speedup 1.0000x reference)
import functools

import jax
import jax.numpy as jnp
from jax import lax
from jax.experimental import pallas as pl
from jax.experimental.pallas import tpu as pltpu
from jax.experimental.pallas import tpu_sc as plsc

N = 10000
E = 320000
D = 128

NC = 2
NS = 16
NW = NC * NS

N_PAD = 10240
K = 128
CHUNKS = (E + NW * K - 1) // (NW * K)
E_PAD = NW * CHUNKS * K

ROWS_PER_TILE = N_PAD // NS
COPY_STEPS = ROWS_PER_TILE // K

_MESH = plsc.VectorSubcoreMesh(core_axis_name="c", subcore_axis_name="s")


@functools.partial(
    pl.kernel,
    out_type=jax.ShapeDtypeStruct((2, NW, N_PAD), jnp.float32),
    mesh=_MESH,
    scratch_types=[
        pltpu.VMEM((CHUNKS, K), jnp.int32),
        pltpu.VMEM((CHUNKS, K), jnp.int32),
        pltpu.VMEM((N_PAD,), jnp.float32),
        pltpu.VMEM((N_PAD,), jnp.float32),
    ],
)
def _degree_kernel(src_hbm, dst_hbm, out_hbm, src_v, dst_v, ci_v, co_v):
    c = lax.axis_index("c")
    s = lax.axis_index("s")
    wid = s * NC + c

    zeros16 = jnp.zeros((16,), jnp.float32)

    def zero_body(i, _):
        ci_v[pl.ds(i * 16, 16)] = zeros16
        co_v[pl.ds(i * 16, 16)] = zeros16
        return 0

    lax.fori_loop(0, N_PAD // 16, zero_body, 0)

    pltpu.sync_copy(src_hbm.at[wid], src_v)
    pltpu.sync_copy(dst_hbm.at[wid], dst_v)

    ones16 = jnp.ones((16,), jnp.float32)

    def body(r, _):
        for g in range(K // 16):
            si = src_v[r, pl.ds(g * 16, 16)]
            di = dst_v[r, pl.ds(g * 16, 16)]
            plsc.addupdate_scatter(co_v, [si], ones16)
            plsc.addupdate_scatter(ci_v, [di], ones16)
        return 0

    lax.fori_loop(0, CHUNKS, body, 0)

    pltpu.sync_copy(ci_v, out_hbm.at[0, wid])
    pltpu.sync_copy(co_v, out_hbm.at[1, wid])


@functools.partial(
    pl.kernel,
    out_type=jax.ShapeDtypeStruct((NC, N_PAD, D), jnp.float32),
    mesh=_MESH,
    scratch_types=[
        pltpu.VMEM((CHUNKS, K), jnp.int32),
        pltpu.VMEM((CHUNKS, K), jnp.int32),
        pltpu.VMEM((K, D), jnp.float32),
        pltpu.VMEM_SHARED((N_PAD, D), jnp.float32),
        pltpu.SemaphoreType.DMA,
    ],
)
def _agg_kernel(h_hbm, gidx_hbm, sidx_hbm, out_hbm, gidx_v, sidx_v, rows_v,
                acc_sh, sem):
    c = lax.axis_index("c")
    s = lax.axis_index("s")
    wid = s * NC + c

    zeros16 = jnp.zeros((16,), jnp.float32)

    def zero_body(r, _):
        for g in range(D // 16):
            rows_v[r, pl.ds(g * 16, 16)] = zeros16
        return 0

    lax.fori_loop(0, K, zero_body, 0)
    for t in range(COPY_STEPS):
        pltpu.sync_copy(rows_v, acc_sh.at[pl.ds(s * ROWS_PER_TILE + t * K, K)])
    plsc.subcore_barrier()

    pltpu.sync_copy(gidx_hbm.at[wid], gidx_v)
    pltpu.sync_copy(sidx_hbm.at[wid], sidx_v)

    def body(j, _):
        pltpu.async_copy(h_hbm.at[gidx_v.at[j]], rows_v, sem).wait()
        pltpu.sync_copy(rows_v, acc_sh.at[sidx_v.at[j]], add=True)
        return 0

    lax.fori_loop(0, CHUNKS, body, 0)
    plsc.subcore_barrier()

    pltpu.sync_copy(acc_sh.at[pl.ds(s * ROWS_PER_TILE, ROWS_PER_TILE)],
                    out_hbm.at[c, pl.ds(s * ROWS_PER_TILE, ROWS_PER_TILE)])


def _linear_body(x_ref, w_ref, b_ref, o_ref):
    o_ref[...] = (
        jnp.dot(x_ref[...], w_ref[...], preferred_element_type=jnp.float32)
        + b_ref[...]
    )


_BLK = 512
_GRID = N_PAD // _BLK


def _tc_linear(x, wt, b2d):
    return pl.pallas_call(
        _linear_body,
        grid=(_GRID,),
        in_specs=[
            pl.BlockSpec((_BLK, D), lambda i: (i, 0)),
            pl.BlockSpec((D, D), lambda i: (0, 0)),
            pl.BlockSpec((1, D), lambda i: (0, 0)),
        ],
        out_specs=pl.BlockSpec((_BLK, D), lambda i: (i, 0)),
        out_shape=jax.ShapeDtypeStruct((N_PAD, D), jnp.float32),
    )(x, wt, b2d)


def _combine1_body(p_ref, h_ref, c_ref, w_ref, b_ref, o_ref):
    deg = jnp.sum(c_ref[...], axis=1, keepdims=True) + 1.0
    agg = (p_ref[0] + p_ref[1] + h_ref[...]) / deg
    y = jnp.maximum(agg, 0.0)
    o_ref[...] = (
        jnp.dot(y, w_ref[...], preferred_element_type=jnp.float32) + b_ref[...]
    )


def _tc_combine_linear(p, h, cnt_t, wt, b2d):
    return pl.pallas_call(
        _combine1_body,
        grid=(_GRID,),
        in_specs=[
            pl.BlockSpec((NC, _BLK, D), lambda i: (0, i, 0)),
            pl.BlockSpec((_BLK, D), lambda i: (i, 0)),
            pl.BlockSpec((_BLK, NW), lambda i: (i, 0)),
            pl.BlockSpec((D, D), lambda i: (0, 0)),
            pl.BlockSpec((1, D), lambda i: (0, 0)),
        ],
        out_specs=pl.BlockSpec((_BLK, D), lambda i: (i, 0)),
        out_shape=jax.ShapeDtypeStruct((N_PAD, D), jnp.float32),
    )(p, h, cnt_t, wt, b2d)


def _combine2_body(p_ref, h_ref, c_ref, o_ref):
    deg = jnp.sum(c_ref[...], axis=1, keepdims=True) + 1.0
    o_ref[...] = (p_ref[0] + p_ref[1] + h_ref[...]) / deg


def _tc_combine(p, h, cnt_t):
    return pl.pallas_call(
        _combine2_body,
        grid=(_GRID,),
        in_specs=[
            pl.BlockSpec((NC, _BLK, D), lambda i: (0, i, 0)),
            pl.BlockSpec((_BLK, D), lambda i: (i, 0)),
            pl.BlockSpec((_BLK, NW), lambda i: (i, 0)),
        ],
        out_specs=pl.BlockSpec((_BLK, D), lambda i: (i, 0)),
        out_shape=jax.ShapeDtypeStruct((N_PAD, D), jnp.float32),
    )(p, h, cnt_t)


def kernel(x, edge_index, W1, b1, W2, b2):
    x_pad = jnp.zeros((N_PAD, D), jnp.float32).at[:N].set(x)

    pad_val = jnp.int32(N_PAD - 1)
    src = jnp.full((E_PAD,), pad_val, jnp.int32).at[:E].set(edge_index[0])
    dst = jnp.full((E_PAD,), pad_val, jnp.int32).at[:E].set(edge_index[1])
    src_r = src.reshape(NW, CHUNKS, K)
    dst_r = dst.reshape(NW, CHUNKS, K)

    cnts = _degree_kernel(src_r, dst_r)
    cin_t = cnts[0].T
    cout_t = cnts[1].T

    h1 = _tc_linear(x_pad, W1.T, b1.reshape(1, D))
    p1 = _agg_kernel(h1, src_r, dst_r)
    h2 = _tc_combine_linear(p1, h1, cin_t, W2.T, b2.reshape(1, D))
    p2 = _agg_kernel(h2, dst_r, src_r)
    out = _tc_combine(p2, h2, cout_t)
    return out[:N]


# speedup vs baseline: 4.2563x; 4.2563x over previous
import functools

import jax
import jax.numpy as jnp
from jax.experimental import pallas as pl

N = 10000
N_PAD = 10240
D = 128

_BLK = 512
_GRID = N_PAD // _BLK


def _linear_body(x_ref, w_ref, b_ref, o_ref):
    o_ref[...] = (
        jnp.dot(x_ref[...], w_ref[...], preferred_element_type=jnp.float32)
        + b_ref[...]
    )


def _tc_linear(x, wt, b2d):
    return pl.pallas_call(
        _linear_body,
        grid=(_GRID,),
        in_specs=[
            pl.BlockSpec((_BLK, D), lambda i: (i, 0)),
            pl.BlockSpec((D, D), lambda i: (0, 0)),
            pl.BlockSpec((1, D), lambda i: (0, 0)),
        ],
        out_specs=pl.BlockSpec((_BLK, D), lambda i: (i, 0)),
        out_shape=jax.ShapeDtypeStruct((N_PAD, D), jnp.float32),
    )(x, wt, b2d)


def _combine1_body(p_ref, d_ref, h_ref, w_ref, b_ref, o_ref):
    deg = d_ref[...] + 1.0
    y = jnp.maximum((p_ref[...] + h_ref[...]) / deg, 0.0)
    o_ref[...] = (
        jnp.dot(y, w_ref[...], preferred_element_type=jnp.float32) + b_ref[...]
    )


def _tc_combine_linear(p, deg, h, wt, b2d):
    return pl.pallas_call(
        _combine1_body,
        grid=(_GRID,),
        in_specs=[
            pl.BlockSpec((_BLK, D), lambda i: (i, 0)),
            pl.BlockSpec((_BLK, 1), lambda i: (i, 0)),
            pl.BlockSpec((_BLK, D), lambda i: (i, 0)),
            pl.BlockSpec((D, D), lambda i: (0, 0)),
            pl.BlockSpec((1, D), lambda i: (0, 0)),
        ],
        out_specs=pl.BlockSpec((_BLK, D), lambda i: (i, 0)),
        out_shape=jax.ShapeDtypeStruct((N_PAD, D), jnp.float32),
    )(p, deg, h, wt, b2d)


def _combine2_body(p_ref, d_ref, h_ref, o_ref):
    o_ref[...] = (p_ref[...] + h_ref[...]) / (d_ref[...] + 1.0)


def _tc_combine(p, deg, h):
    return pl.pallas_call(
        _combine2_body,
        grid=(_GRID,),
        in_specs=[
            pl.BlockSpec((_BLK, D), lambda i: (i, 0)),
            pl.BlockSpec((_BLK, 1), lambda i: (i, 0)),
            pl.BlockSpec((_BLK, D), lambda i: (i, 0)),
        ],
        out_specs=pl.BlockSpec((_BLK, D), lambda i: (i, 0)),
        out_shape=jax.ShapeDtypeStruct((N_PAD, D), jnp.float32),
    )(p, deg, h)


def kernel(x, edge_index, W1, b1, W2, b2):
    src, dst = edge_index[0], edge_index[1]
    ones = jnp.ones(src.shape, jnp.float32)
    deg_in = jax.ops.segment_sum(ones, dst, num_segments=N_PAD)[:, None]
    deg_out = jax.ops.segment_sum(ones, src, num_segments=N_PAD)[:, None]

    x_pad = jnp.zeros((N_PAD, D), jnp.float32).at[:N].set(x)
    h1 = _tc_linear(x_pad, W1.T, b1.reshape(1, D))
    agg1 = jax.ops.segment_sum(h1[src], dst, num_segments=N_PAD)
    h2 = _tc_combine_linear(agg1, deg_in, h1, W2.T, b2.reshape(1, D))
    agg2 = jax.ops.segment_sum(h2[dst], src, num_segments=N_PAD)
    return _tc_combine(agg2, deg_out, h2)[:N]
